# baseline (speedup 1.0000x reference)
import functools

import jax
import jax.numpy as jnp
from jax import lax
from jax.experimental import pallas as pl
from jax.experimental.pallas import tpu as pltpu
from jax.experimental.pallas import tpu_sc as plsc

N_CB = 2
K = 1024
D = 128
DM = 256
ROWS = 8192
RB = 512
NB = ROWS // RB

_NC = 2
_NS = 16
_NW = _NC * _NS
_BPW = ROWS // _NW
_CHUNK = 128
_NCH = _BPW // _CHUNK


def _argmin_body(z_ref, cb_ref, w_ref, idx_ref, dsum_ref, m_ref):
    zc = z_ref[...]
    cb = cb_ref[0]

    @pl.when(pl.program_id(0) == 0)
    def _build_table():
        m_ref[...] = jnp.dot(cb, w_ref[0],
                             preferred_element_type=jnp.float32)

    s = lax.dot_general(cb, zc, (((1,), (1,)), ((), ())),
                        preferred_element_type=jnp.float32)
    cn = jnp.sum(cb * cb, axis=1)
    d = cn[:, None] - 2.0 * s
    part = d[0:128, :]
    for j in range(1, K // 128):
        part = jnp.minimum(part, d[j * 128:(j + 1) * 128, :])
    m = jnp.min(part, axis=0, keepdims=True)
    iota = lax.broadcasted_iota(jnp.int32, (128, RB), 0)
    ip = jnp.full((128, RB), K, jnp.int32)
    for j in range(K // 128):
        blk = d[j * 128:(j + 1) * 128, :]
        ip = jnp.minimum(ip, jnp.where(blk == m, iota + j * 128, K))
    idx_ref[0, 0, :] = jnp.min(ip, axis=0)
    acc = jnp.sum(zc * zc) + jnp.sum(m)
    prev = jnp.where(pl.program_id(0) == 0,
                     jnp.zeros((1, 1), jnp.float32), dsum_ref[...])
    dsum_ref[...] = prev + acc


def _argmin(zf, codebooks, wr, i_cb):
    return pl.pallas_call(
        _argmin_body,
        grid=(NB,),
        in_specs=[
            pl.BlockSpec((RB, D), lambda r: (r, i_cb)),
            pl.BlockSpec((1, K, D), lambda r: (i_cb, 0, 0)),
            pl.BlockSpec((1, D, DM), lambda r: (i_cb, 0, 0)),
        ],
        out_specs=[
            pl.BlockSpec((1, 1, RB), lambda r: (r, 0, 0)),
            pl.BlockSpec((1, 1), lambda r: (0, 0)),
            pl.BlockSpec((K, DM), lambda r: (0, 0)),
        ],
        out_shape=[
            jax.ShapeDtypeStruct((NB, 1, RB), jnp.int32),
            jax.ShapeDtypeStruct((1, 1), jnp.float32),
            jax.ShapeDtypeStruct((K, DM), jnp.float32),
        ],
    )(zf, codebooks, wr)


def _sc_gather(table, idx):
    mesh = plsc.VectorSubcoreMesh(core_axis_name="c", subcore_axis_name="s")

    @functools.partial(
        pl.kernel, mesh=mesh,
        out_type=jax.ShapeDtypeStruct((ROWS, DM), jnp.float32),
        scratch_types=[
            pltpu.VMEM((_NCH, _CHUNK), jnp.int32),
            pltpu.VMEM((_CHUNK, DM), jnp.float32),
            pltpu.VMEM((_CHUNK, DM), jnp.float32),
            pltpu.SemaphoreType.DMA,
            pltpu.SemaphoreType.DMA,
            pltpu.SemaphoreType.DMA,
        ],
    )
    def k(tab_hbm, i_hbm, out_hbm, i_v, ra, rb, sema, semb, wsem):
        wid = lax.axis_index("s") * _NC + lax.axis_index("c")
        base = wid * _BPW
        pltpu.sync_copy(i_hbm.at[wid], i_v)
        bufs = [(ra, sema), (rb, semb)]

        def fire(c):
            r, sem = bufs[c % 2]
            return pltpu.async_copy(tab_hbm.at[i_v.at[c]], r, sem)

        gh = {0: fire(0)}
        wh = {}
        for c in range(_NCH):
            r, _ = bufs[c % 2]
            if c + 1 < _NCH:
                gh[c + 1] = fire(c + 1)
            gh[c].wait()
            wh[c] = pltpu.async_copy(
                r, out_hbm.at[pl.ds(base + c * _CHUNK, _CHUNK)], wsem)
        for c in range(_NCH):
            wh[c].wait()

    return k(table, idx)


def _add_body(a_ref, b_ref, o_ref):
    o_ref[...] = a_ref[...] + b_ref[...]


def _tc_add(a, b):
    blk = 1024
    return pl.pallas_call(
        _add_body,
        grid=(ROWS // blk,),
        in_specs=[
            pl.BlockSpec((blk, DM), lambda r: (r, 0)),
            pl.BlockSpec((blk, DM), lambda r: (r, 0)),
        ],
        out_specs=pl.BlockSpec((blk, DM), lambda r: (r, 0)),
        out_shape=jax.ShapeDtypeStruct((ROWS, DM), jnp.float32),
    )(a, b)


def kernel(z, codebooks, W_dec):
    zf = z.reshape(ROWS, DM)
    wr = W_dec.reshape(N_CB, D, DM)
    idx0, dsum0, m0 = _argmin(zf, codebooks, wr, 0)
    part = _sc_gather(m0, idx0.reshape(_NW, _NCH, _CHUNK))
    idx1, dsum1, m1 = _argmin(zf, codebooks, wr, 1)
    g1 = _sc_gather(m1, idx1.reshape(_NW, _NCH, _CHUNK))
    out = _tc_add(part, g1)
    output = out.reshape(z.shape)
    diff_mean = (dsum0[0, 0] + dsum1[0, 0]) * (1.0 / (N_CB * ROWS * D))
    return output, diff_mean


# speedup vs baseline: 1.3662x; 1.3662x over previous
import functools

import jax
import jax.numpy as jnp
from jax import lax
from jax.experimental import pallas as pl
from jax.experimental.pallas import tpu as pltpu
from jax.experimental.pallas import tpu_sc as plsc

N_CB = 2
K = 1024
D = 128
DM = 256
ROWS = 8192
RB = 512
NB = ROWS // RB

_NC = 2
_NS = 16
_NW = _NC * _NS
_BPW = ROWS // _NW
_CHUNK = 64


def _tables_body(cb_ref, w_ref, m_ref):
    m_ref[0] = jnp.dot(cb_ref[0], w_ref[...],
                       preferred_element_type=jnp.float32)


def _build_tables(codebooks, W_dec):
    return pl.pallas_call(
        _tables_body,
        grid=(N_CB,),
        in_specs=[
            pl.BlockSpec((1, K, D), lambda i: (i, 0, 0)),
            pl.BlockSpec((D, DM), lambda i: (i, 0)),
        ],
        out_specs=pl.BlockSpec((1, K, DM), lambda i: (i, 0, 0)),
        out_shape=jax.ShapeDtypeStruct((N_CB, K, DM), jnp.float32),
    )(codebooks, W_dec)


def _argmin_body(z_ref, cb_ref, idx0_ref, idx1_ref, dsum_ref):
    zb = z_ref[...]
    acc = jnp.float32(0.0)
    for i in range(N_CB):
        cb = cb_ref[i]
        zc = zb[:, i * D:(i + 1) * D]
        s = lax.dot_general(cb, zc, (((1,), (1,)), ((), ())),
                            preferred_element_type=jnp.float32)
        cn = jnp.sum(cb * cb, axis=1)
        d = cn[:, None] - 2.0 * s
        iota = lax.broadcasted_iota(jnp.int32, (128, RB), 0)
        best = d[0:128, :]
        bidx = iota
        for j in range(1, K // 128):
            blk = d[j * 128:(j + 1) * 128, :]
            lt = blk < best
            best = jnp.minimum(best, blk)
            bidx = jnp.where(lt, iota + j * 128, bidx)
        m = jnp.min(best, axis=0, keepdims=True)
        idx = jnp.min(jnp.where(best == m, bidx, K), axis=0)
        if i == 0:
            idx0_ref[0, 0, :] = idx
        else:
            idx1_ref[0, 0, :] = idx + K
        acc += jnp.sum(zc * zc) + jnp.sum(m)
    prev = jnp.where(pl.program_id(0) == 0,
                     jnp.zeros((1, 1), jnp.float32), dsum_ref[...])
    dsum_ref[...] = prev + acc


def _argmin(zf, codebooks):
    return pl.pallas_call(
        _argmin_body,
        grid=(NB,),
        in_specs=[
            pl.BlockSpec((RB, DM), lambda i: (i, 0)),
            pl.BlockSpec((N_CB, K, D), lambda i: (0, 0, 0)),
        ],
        out_specs=[
            pl.BlockSpec((1, 1, RB), lambda i: (i, 0, 0)),
            pl.BlockSpec((1, 1, RB), lambda i: (i, 0, 0)),
            pl.BlockSpec((1, 1), lambda i: (0, 0)),
        ],
        out_shape=[
            jax.ShapeDtypeStruct((NB, 1, RB), jnp.int32),
            jax.ShapeDtypeStruct((NB, 1, RB), jnp.int32),
            jax.ShapeDtypeStruct((1, 1), jnp.float32),
        ],
    )(zf, codebooks)


def _sc_gather_add(tables, idxc):
    mesh = plsc.VectorSubcoreMesh(core_axis_name="c", subcore_axis_name="s")
    nchunks = _BPW // _CHUNK

    @functools.partial(
        pl.kernel, mesh=mesh,
        out_type=jax.ShapeDtypeStruct((ROWS, DM), jnp.float32),
        scratch_types=[
            pltpu.VMEM((nchunks, 2 * _CHUNK), jnp.int32),
            pltpu.VMEM((2 * _CHUNK, DM), jnp.float32),
            pltpu.VMEM((2 * _CHUNK, DM), jnp.float32),
            pltpu.SemaphoreType.DMA,
            pltpu.SemaphoreType.DMA,
            pltpu.SemaphoreType.DMA,
        ],
    )
    def k(tab_hbm, ic_hbm, out_hbm, ic_v, ra, rb, sema, semb, wsem):
        wid = lax.axis_index("s") * _NC + lax.axis_index("c")
        base = wid * _BPW
        pltpu.sync_copy(ic_hbm.at[wid], ic_v)
        bufs = [(ra, sema), (rb, semb)]

        def fire(c):
            r, sem = bufs[c % 2]
            return pltpu.async_copy(tab_hbm.at[ic_v.at[c]], r, sem)

        gh = {0: fire(0)}
        wh = {}
        for c in range(nchunks):
            r, _ = bufs[c % 2]
            if c + 1 < nchunks:
                if c - 1 >= 0:
                    wh[c - 1].wait()
                gh[c + 1] = fire(c + 1)
            gh[c].wait()

            def body(i, carry):
                for g in range(DM // 16):
                    sl = pl.ds(g * 16, 16)
                    r[i, sl] = r[i, sl] + r[_CHUNK + i, sl]
                return carry

            lax.fori_loop(0, _CHUNK, body, 0)
            wh[c] = pltpu.async_copy(
                r.at[pl.ds(0, _CHUNK)],
                out_hbm.at[pl.ds(base + c * _CHUNK, _CHUNK)], wsem)
        wh[nchunks - 2].wait()
        wh[nchunks - 1].wait()

    return k(tables, idxc)


def kernel(z, codebooks, W_dec):
    zf = z.reshape(ROWS, DM)
    tables = _build_tables(codebooks, W_dec).reshape(N_CB * K, DM)
    idx0, idx1, dsum = _argmin(zf, codebooks)
    nchunks = _BPW // _CHUNK
    i0 = idx0.reshape(_NW, nchunks, 1, _CHUNK)
    i1 = idx1.reshape(_NW, nchunks, 1, _CHUNK)
    idxc = jnp.concatenate([i0, i1], axis=2).reshape(_NW, nchunks, 2 * _CHUNK)
    out = _sc_gather_add(tables, idxc)
    output = out.reshape(z.shape)
    diff_mean = dsum[0, 0] * (1.0 / (N_CB * ROWS * D))
    return output, diff_mean
